# baseline (speedup 1.0000x reference)
import functools

import jax
import jax.numpy as jnp
from jax import lax
from jax.experimental import pallas as pl
from jax.experimental.pallas import tpu as pltpu
from jax.experimental.pallas import tpu_sc as plsc

N = 10000
NP = 10240
D = 256
HALF = 128
NCLS = 6
CPAD = 16
CHUNK = 128
NC, NS = 2, 16
NW = NC * NS
E_PAD = 163840
CH_W = E_PAD // (NW * CHUNK)
CH_T = E_PAD // (NS * CHUNK)
DUMP = N
RPT = NP // NS
BLK = 1024
NBUF = 4


def _mesh():
    return plsc.VectorSubcoreMesh(core_axis_name="c", subcore_axis_name="s")


def _sc_deg(dst4, ones_h, zeros_h):
    @functools.partial(
        pl.kernel, mesh=_mesh(),
        out_type=jax.ShapeDtypeStruct((NC, NP, HALF), jnp.float32),
        scratch_types=[
            pltpu.VMEM((CH_W, CHUNK), jnp.int32),
            pltpu.VMEM((CHUNK, HALF), jnp.float32),
            pltpu.VMEM_SHARED((NP, HALF), jnp.float32),
        ],
    )
    def k(dst_hbm, ones_hbm, zeros_hbm, out_hbm, dst_v, ones_v, acc):
        cid = lax.axis_index("c")
        sid = lax.axis_index("s")
        wid = sid * NC + cid
        pltpu.sync_copy(zeros_hbm, acc.at[pl.ds(sid * RPT, RPT)])
        pltpu.sync_copy(ones_hbm, ones_v)
        pltpu.sync_copy(dst_hbm.at[wid], dst_v)
        plsc.subcore_barrier()

        @pl.loop(0, CH_W)
        def _(j):
            pltpu.sync_copy(ones_v, acc.at[dst_v.at[j]], add=True)

        plsc.subcore_barrier()
        pltpu.sync_copy(acc.at[pl.ds(sid * RPT, RPT)],
                        out_hbm.at[cid, pl.ds(sid * RPT, RPT)])

    return k(dst4, ones_h, zeros_h)


def _sc_agg(table, src_idx, dst_idx, zeros_h, ch):
    @functools.partial(
        pl.kernel, mesh=_mesh(),
        out_type=jax.ShapeDtypeStruct((NC, NP, HALF), jnp.float32),
        scratch_types=[
            pltpu.VMEM((ch, CHUNK), jnp.int32),
            pltpu.VMEM((CHUNK,), jnp.int32),
            pltpu.VMEM((CHUNK,), jnp.int32),
            pltpu.VMEM((CHUNK, HALF), jnp.float32),
            pltpu.VMEM((CHUNK, HALF), jnp.float32),
            pltpu.SemaphoreType.DMA,
            pltpu.SemaphoreType.DMA,
            pltpu.SemaphoreType.DMA,
            pltpu.SemaphoreType.DMA,
            pltpu.VMEM_SHARED((NP, HALF), jnp.float32),
        ],
    )
    def k(tab_hbm, src_hbm, dst_hbm, zeros_hbm, out_hbm,
          dst_v, s0, s1, r0, r1, si0, si1, sg0, sg1, acc):
        cid = lax.axis_index("c")
        sid = lax.axis_index("s")
        pltpu.sync_copy(zeros_hbm, acc.at[pl.ds(sid * RPT, RPT)])
        pltpu.sync_copy(dst_hbm.at[cid, sid], dst_v)
        plsc.subcore_barrier()

        slots = ((s0, r0, si0, sg0), (s1, r1, si1, sg1))
        pltpu.async_copy(src_hbm.at[cid, sid, 0], s0, si0)
        pltpu.async_copy(src_hbm.at[cid, sid, 1], s1, si1)
        pltpu.make_async_copy(src_hbm.at[cid, sid, 0], s0, si0).wait()
        pltpu.async_copy(tab_hbm.at[s0], r0, sg0)
        pltpu.make_async_copy(src_hbm.at[cid, sid, 0], s1, si1).wait()
        pltpu.async_copy(tab_hbm.at[s1], r1, sg1)

        @pl.loop(0, ch // 2)
        def _(g):
            for b, (sb, rb, sib, sgb) in enumerate(slots):
                j = 2 * g + b
                nxt = j + 2
                pltpu.make_async_copy(tab_hbm.at[sb], rb, sgb).wait()

                @pl.when(nxt < ch)
                def _():
                    pltpu.async_copy(src_hbm.at[cid, sid, nxt], sb, sib)

                pltpu.sync_copy(rb, acc.at[dst_v.at[j]], add=True)

                @pl.when(nxt < ch)
                def _():
                    pltpu.make_async_copy(src_hbm.at[cid, sid, 0], sb, sib).wait()
                    pltpu.async_copy(tab_hbm.at[sb], rb, sgb)

        plsc.subcore_barrier()
        pltpu.sync_copy(acc.at[pl.ds(sid * RPT, RPT)],
                        out_hbm.at[cid, pl.ds(sid * RPT, RPT)])

    return k(table, src_idx, dst_idx, zeros_h)


def _dinv(deg_blk):
    d = deg_blk[0, :, :1] + deg_blk[1, :, :1] + 1.0
    return lax.rsqrt(d)


def _tc_xw(xp, W1):
    def body(x_ref, w_ref, out_ref):
        xw = lax.dot_general(x_ref[...], w_ref[...], (((1,), (0,)), ((), ())),
                             precision=lax.Precision.HIGHEST)
        out_ref[0] = xw[:, :HALF]
        out_ref[1] = xw[:, HALF:]

    return pl.pallas_call(
        body,
        grid=(NP // BLK,),
        in_specs=[pl.BlockSpec((BLK, D), lambda i: (i, 0)),
                  pl.BlockSpec((D, D), lambda i: (0, 0))],
        out_specs=pl.BlockSpec((NC, BLK, HALF), lambda i: (0, i, 0)),
        out_shape=jax.ShapeDtypeStruct((NC, NP, HALF), jnp.float32),
    )(xp, W1)


def _tc_scale(xw, deg):
    def body(xw_ref, deg_ref, out_ref):
        dinv = _dinv(deg_ref[...])
        out_ref[0] = xw_ref[0] * dinv
        out_ref[1] = xw_ref[1] * dinv

    return pl.pallas_call(
        body,
        grid=(NP // BLK,),
        in_specs=[pl.BlockSpec((NC, BLK, HALF), lambda i: (0, i, 0)),
                  pl.BlockSpec((NC, BLK, HALF), lambda i: (0, i, 0))],
        out_specs=pl.BlockSpec((NC, BLK, HALF), lambda i: (0, i, 0)),
        out_shape=jax.ShapeDtypeStruct((NC, NP, HALF), jnp.float32),
    )(xw, deg)


def _tc_hidden(agg, xws, deg, b1r, W2p):
    def body(agg_ref, xws_ref, deg_ref, b1_ref, w2_ref, out_ref):
        a = agg_ref[...]
        s = xws_ref[...]
        full_a = jnp.concatenate([a[0], a[1]], axis=1)
        full_s = jnp.concatenate([s[0], s[1]], axis=1)
        dinv = _dinv(deg_ref[...])
        h = jnp.maximum(dinv * (full_a + full_s) + b1_ref[0], 0.0)
        hw = lax.dot_general(h, w2_ref[...], (((1,), (0,)), ((), ())),
                             precision=lax.Precision.HIGHEST)
        out_ref[...] = dinv * hw

    return pl.pallas_call(
        body,
        grid=(NP // BLK,),
        in_specs=[pl.BlockSpec((NC, BLK, HALF), lambda i: (0, i, 0)),
                  pl.BlockSpec((NC, BLK, HALF), lambda i: (0, i, 0)),
                  pl.BlockSpec((NC, BLK, HALF), lambda i: (0, i, 0)),
                  pl.BlockSpec((1, D), lambda i: (0, 0)),
                  pl.BlockSpec((D, HALF), lambda i: (0, 0))],
        out_specs=pl.BlockSpec((BLK, HALF), lambda i: (i, 0)),
        out_shape=jax.ShapeDtypeStruct((NP, HALF), jnp.float32),
    )(agg, xws, deg, b1r, W2p)


def _tc_final(agg2, hws, deg, b2p):
    def body(agg_ref, hws_ref, deg_ref, b2_ref, out_ref):
        a = agg_ref[...]
        dinv = _dinv(deg_ref[...])
        v = dinv * (a[0] + a[1] + hws_ref[...]) + b2_ref[0]
        col = lax.broadcasted_iota(jnp.int32, (BLK, HALF), 1)
        valid = col < NCLS
        m = jnp.max(jnp.where(valid, v, -1e30), axis=1, keepdims=True)
        ex = jnp.where(valid, jnp.exp(v - m), 0.0)
        ssum = jnp.sum(ex, axis=1, keepdims=True)
        out_ref[...] = v - m - jnp.log(ssum)

    return pl.pallas_call(
        body,
        grid=(NP // BLK,),
        in_specs=[pl.BlockSpec((NC, BLK, HALF), lambda i: (0, i, 0)),
                  pl.BlockSpec((BLK, HALF), lambda i: (i, 0)),
                  pl.BlockSpec((NC, BLK, HALF), lambda i: (0, i, 0)),
                  pl.BlockSpec((1, HALF), lambda i: (0, 0))],
        out_specs=pl.BlockSpec((BLK, HALF), lambda i: (i, 0)),
        out_shape=jax.ShapeDtypeStruct((NP, HALF), jnp.float32),
    )(agg2, hws, deg, b2p)


def kernel(x, edge_index, W1, b1, W2, b2):
    src = edge_index[0].astype(jnp.int32)
    dst = edge_index[1].astype(jnp.int32)
    e = src.shape[0]
    pad = E_PAD - e
    src_p = jnp.concatenate([src, jnp.zeros((pad,), jnp.int32)])
    dst_p = jnp.concatenate([dst, jnp.full((pad,), DUMP, jnp.int32)])

    dst4 = dst_p.reshape(NW, CH_W, CHUNK)
    src_w = src_p.reshape(NC, NS, CH_W, CHUNK)
    dst_w = dst_p.reshape(NC, NS, CH_W, CHUNK)
    src_t = jnp.stack([src_p, src_p + NP]).reshape(NC, NS, CH_T, CHUNK)
    dst_t = jnp.broadcast_to(dst_p.reshape(1, NS, CH_T, CHUNK),
                             (NC, NS, CH_T, CHUNK))

    xp = jnp.pad(x, ((0, NP - N), (0, 0)))
    zeros_h = jnp.zeros((RPT, HALF), jnp.float32)
    ones_h = jnp.ones((CHUNK, HALF), jnp.float32)
    b1r = b1.reshape(1, D)
    W2p = jnp.pad(W2, ((0, 0), (0, HALF - NCLS)))
    b2p = jnp.pad(b2, (0, HALF - NCLS)).reshape(1, HALF)

    deg = _sc_deg(dst4, ones_h, zeros_h)
    xw = _tc_xw(xp, W1)
    xws = _tc_scale(xw, deg)
    table = xws.reshape(NC * NP, HALF)
    agg = _sc_agg(table, src_t, dst_t, zeros_h, CH_T)
    hws = _tc_hidden(agg, xws, deg, b1r, W2p)
    agg2 = _sc_agg(hws, src_w, dst_w, zeros_h, CH_W)
    outw = _tc_final(agg2, hws, deg, b2p)
    return outw[:N, :NCLS]


# speedup vs baseline: 1.0365x; 1.0365x over previous
import functools

import jax
import jax.numpy as jnp
from jax import lax
from jax.experimental import pallas as pl
from jax.experimental.pallas import tpu as pltpu
from jax.experimental.pallas import tpu_sc as plsc

N = 10000
NP = 10240
D = 256
HALF = 128
NCLS = 6
CPAD = 16
CHUNK = 64
DCHUNK = 128
NC, NS = 2, 16
NW = NC * NS
E_PAD = 163840
CH_W = E_PAD // (NW * CHUNK)
CH_T = E_PAD // (NS * CHUNK)
CH_D = E_PAD // (NW * DCHUNK)
DUMP = N
RPT = NP // NS
BLK = 1024
NBUF = 4


def _mesh():
    return plsc.VectorSubcoreMesh(core_axis_name="c", subcore_axis_name="s")


def _sc_deg(dst4, ones_h, zeros_h):
    @functools.partial(
        pl.kernel, mesh=_mesh(),
        out_type=jax.ShapeDtypeStruct((NC, NP, HALF), jnp.float32),
        scratch_types=[
            pltpu.VMEM((CH_D, DCHUNK), jnp.int32),
            pltpu.VMEM((DCHUNK, HALF), jnp.float32),
            pltpu.VMEM_SHARED((NP, HALF), jnp.float32),
        ],
    )
    def k(dst_hbm, ones_hbm, zeros_hbm, out_hbm, dst_v, ones_v, acc):
        cid = lax.axis_index("c")
        sid = lax.axis_index("s")
        wid = sid * NC + cid
        pltpu.sync_copy(zeros_hbm, acc.at[pl.ds(sid * RPT, RPT)])
        pltpu.sync_copy(ones_hbm, ones_v)
        pltpu.sync_copy(dst_hbm.at[wid], dst_v)
        plsc.subcore_barrier()

        @pl.loop(0, CH_D)
        def _(j):
            pltpu.sync_copy(ones_v, acc.at[dst_v.at[j]], add=True)

        plsc.subcore_barrier()
        pltpu.sync_copy(acc.at[pl.ds(sid * RPT, RPT)],
                        out_hbm.at[cid, pl.ds(sid * RPT, RPT)])

    return k(dst4, ones_h, zeros_h)


def _sc_agg(table, src_idx, dst_idx, zeros_h, ch):
    @functools.partial(
        pl.kernel, mesh=_mesh(),
        out_type=jax.ShapeDtypeStruct((NC, NP, HALF), jnp.float32),
        scratch_types=[pltpu.VMEM((CHUNK,), jnp.int32) for _ in range(NBUF)]
        + [pltpu.VMEM((CHUNK,), jnp.int32) for _ in range(NBUF)]
        + [pltpu.VMEM((CHUNK, HALF), jnp.float32) for _ in range(NBUF)]
        + [pltpu.SemaphoreType.DMA for _ in range(3 * NBUF)]
        + [pltpu.VMEM_SHARED((NP, HALF), jnp.float32)],
    )
    def k(tab_hbm, src_hbm, dst_hbm, zeros_hbm, out_hbm, *rest):
        sidx = rest[:NBUF]
        didx = rest[NBUF:2 * NBUF]
        rows = rest[2 * NBUF:3 * NBUF]
        sis = rest[3 * NBUF:4 * NBUF]
        sds = rest[4 * NBUF:5 * NBUF]
        sgs = rest[5 * NBUF:6 * NBUF]
        acc = rest[6 * NBUF]
        cid = lax.axis_index("c")
        sid = lax.axis_index("s")
        pltpu.sync_copy(zeros_hbm, acc.at[pl.ds(sid * RPT, RPT)])
        plsc.subcore_barrier()

        for b in range(NBUF):
            pltpu.async_copy(src_hbm.at[cid, sid, b], sidx[b], sis[b])
            pltpu.async_copy(dst_hbm.at[cid, sid, b], didx[b], sds[b])
        for b in range(NBUF):
            pltpu.make_async_copy(src_hbm.at[cid, sid, 0], sidx[b], sis[b]).wait()
            pltpu.async_copy(tab_hbm.at[sidx[b]], rows[b], sgs[b])

        @pl.loop(0, ch // NBUF)
        def _(g):
            for b in range(NBUF):
                j = g * NBUF + b
                nxt = j + NBUF
                pltpu.make_async_copy(tab_hbm.at[sidx[b]], rows[b],
                                      sgs[b]).wait()

                @pl.when(nxt < ch)
                def _():
                    pltpu.async_copy(src_hbm.at[cid, sid, nxt], sidx[b], sis[b])

                pltpu.make_async_copy(dst_hbm.at[cid, sid, 0], didx[b],
                                      sds[b]).wait()
                pltpu.sync_copy(rows[b], acc.at[didx[b]], add=True)

                @pl.when(nxt < ch)
                def _():
                    pltpu.async_copy(dst_hbm.at[cid, sid, nxt], didx[b], sds[b])
                    pltpu.make_async_copy(src_hbm.at[cid, sid, 0], sidx[b],
                                          sis[b]).wait()
                    pltpu.async_copy(tab_hbm.at[sidx[b]], rows[b], sgs[b])

        plsc.subcore_barrier()
        pltpu.sync_copy(acc.at[pl.ds(sid * RPT, RPT)],
                        out_hbm.at[cid, pl.ds(sid * RPT, RPT)])

    return k(table, src_idx, dst_idx, zeros_h)


def _dinv(deg_blk):
    d = deg_blk[0, :, :1] + deg_blk[1, :, :1] + 1.0
    return lax.rsqrt(d)


def _tc_xws(xp, W1, deg):
    def body(x_ref, w_ref, deg_ref, out_ref):
        xw = lax.dot_general(x_ref[...], w_ref[...], (((1,), (0,)), ((), ())),
                             precision=lax.Precision.HIGHEST)
        xws = xw * _dinv(deg_ref[...])
        out_ref[0] = xws[:, :HALF]
        out_ref[1] = xws[:, HALF:]

    return pl.pallas_call(
        body,
        grid=(NP // BLK,),
        in_specs=[pl.BlockSpec((BLK, D), lambda i: (i, 0)),
                  pl.BlockSpec((D, D), lambda i: (0, 0)),
                  pl.BlockSpec((NC, BLK, HALF), lambda i: (0, i, 0))],
        out_specs=pl.BlockSpec((NC, BLK, HALF), lambda i: (0, i, 0)),
        out_shape=jax.ShapeDtypeStruct((NC, NP, HALF), jnp.float32),
    )(xp, W1, deg)


def _tc_hidden(agg, xws, deg, b1r, W2p):
    def body(agg_ref, xws_ref, deg_ref, b1_ref, w2_ref, out_ref):
        a = agg_ref[...]
        s = xws_ref[...]
        full_a = jnp.concatenate([a[0], a[1]], axis=1)
        full_s = jnp.concatenate([s[0], s[1]], axis=1)
        dinv = _dinv(deg_ref[...])
        h = jnp.maximum(dinv * (full_a + full_s) + b1_ref[0], 0.0)
        hw = lax.dot_general(h, w2_ref[...], (((1,), (0,)), ((), ())),
                             precision=lax.Precision.HIGHEST)
        out_ref[...] = dinv * hw

    return pl.pallas_call(
        body,
        grid=(NP // BLK,),
        in_specs=[pl.BlockSpec((NC, BLK, HALF), lambda i: (0, i, 0)),
                  pl.BlockSpec((NC, BLK, HALF), lambda i: (0, i, 0)),
                  pl.BlockSpec((NC, BLK, HALF), lambda i: (0, i, 0)),
                  pl.BlockSpec((1, D), lambda i: (0, 0)),
                  pl.BlockSpec((D, HALF), lambda i: (0, 0))],
        out_specs=pl.BlockSpec((BLK, HALF), lambda i: (i, 0)),
        out_shape=jax.ShapeDtypeStruct((NP, HALF), jnp.float32),
    )(agg, xws, deg, b1r, W2p)


def _tc_final(agg2, hws, deg, b2p):
    def body(agg_ref, hws_ref, deg_ref, b2_ref, out_ref):
        a = agg_ref[...]
        dinv = _dinv(deg_ref[...])
        v = dinv * (a[0] + a[1] + hws_ref[...]) + b2_ref[0]
        col = lax.broadcasted_iota(jnp.int32, (BLK, HALF), 1)
        valid = col < NCLS
        m = jnp.max(jnp.where(valid, v, -1e30), axis=1, keepdims=True)
        ex = jnp.where(valid, jnp.exp(v - m), 0.0)
        ssum = jnp.sum(ex, axis=1, keepdims=True)
        out_ref[...] = v - m - jnp.log(ssum)

    return pl.pallas_call(
        body,
        grid=(NP // BLK,),
        in_specs=[pl.BlockSpec((NC, BLK, HALF), lambda i: (0, i, 0)),
                  pl.BlockSpec((BLK, HALF), lambda i: (i, 0)),
                  pl.BlockSpec((NC, BLK, HALF), lambda i: (0, i, 0)),
                  pl.BlockSpec((1, HALF), lambda i: (0, 0))],
        out_specs=pl.BlockSpec((BLK, HALF), lambda i: (i, 0)),
        out_shape=jax.ShapeDtypeStruct((NP, HALF), jnp.float32),
    )(agg2, hws, deg, b2p)


def kernel(x, edge_index, W1, b1, W2, b2):
    src = edge_index[0].astype(jnp.int32)
    dst = edge_index[1].astype(jnp.int32)
    e = src.shape[0]
    pad = E_PAD - e
    src_p = jnp.concatenate([src, jnp.zeros((pad,), jnp.int32)])
    dst_p = jnp.concatenate([dst, jnp.full((pad,), DUMP, jnp.int32)])

    dst4 = dst_p.reshape(NW, CH_D, DCHUNK)
    src_w = src_p.reshape(NC, NS, CH_W, CHUNK)
    dst_w = dst_p.reshape(NC, NS, CH_W, CHUNK)
    src_t = jnp.stack([src_p, src_p + NP]).reshape(NC, NS, CH_T, CHUNK)
    dst_t = jnp.broadcast_to(dst_p.reshape(1, NS, CH_T, CHUNK),
                             (NC, NS, CH_T, CHUNK))

    xp = jnp.pad(x, ((0, NP - N), (0, 0)))
    zeros_h = jnp.zeros((RPT, HALF), jnp.float32)
    ones_h = jnp.ones((DCHUNK, HALF), jnp.float32)
    b1r = b1.reshape(1, D)
    W2p = jnp.pad(W2, ((0, 0), (0, HALF - NCLS)))
    b2p = jnp.pad(b2, (0, HALF - NCLS)).reshape(1, HALF)

    deg = _sc_deg(dst4, ones_h, zeros_h)
    xws = _tc_xws(xp, W1, deg)
    table = xws.reshape(NC * NP, HALF)
    agg = _sc_agg(table, src_t, dst_t, zeros_h, CH_T)
    hws = _tc_hidden(agg, xws, deg, b1r, W2p)
    agg2 = _sc_agg(hws, src_w, dst_w, zeros_h, CH_W)
    outw = _tc_final(agg2, hws, deg, b2p)
    return outw[:N, :NCLS]


# speedup vs baseline: 1.1275x; 1.0878x over previous
import functools

import jax
import jax.numpy as jnp
from jax import lax
from jax.experimental import pallas as pl
from jax.experimental.pallas import tpu as pltpu
from jax.experimental.pallas import tpu_sc as plsc

N = 10000
NP = 10240
D = 256
HALF = 128
NCLS = 6
CPAD = 16
CHUNK = 128
DCHUNK = 128
NC, NS = 2, 16
NW = NC * NS
E_PAD = 163840
CH_W = E_PAD // (NW * CHUNK)
CH_T = E_PAD // (NS * CHUNK)
CH_D = E_PAD // (NW * DCHUNK)
DUMP = N
RPT = NP // NS
N_ACC = 10112
RPT_A = N_ACC // NS
BLK = 1024
NBUF = 3


def _mesh():
    return plsc.VectorSubcoreMesh(core_axis_name="c", subcore_axis_name="s")


def _sc_deg(dst4, ones_h, zeros_h):
    @functools.partial(
        pl.kernel, mesh=_mesh(),
        out_type=jax.ShapeDtypeStruct((NC, NP, HALF), jnp.float32),
        scratch_types=[
            pltpu.VMEM((CH_D, DCHUNK), jnp.int32),
            pltpu.VMEM((DCHUNK, HALF), jnp.float32),
            pltpu.VMEM_SHARED((NP, HALF), jnp.float32),
        ],
    )
    def k(dst_hbm, ones_hbm, zeros_hbm, out_hbm, dst_v, ones_v, acc):
        cid = lax.axis_index("c")
        sid = lax.axis_index("s")
        wid = sid * NC + cid
        pltpu.sync_copy(zeros_hbm, acc.at[pl.ds(sid * RPT, RPT)])
        pltpu.sync_copy(ones_hbm, ones_v)
        pltpu.sync_copy(dst_hbm.at[wid], dst_v)
        plsc.subcore_barrier()

        @pl.loop(0, CH_D)
        def _(j):
            pltpu.sync_copy(ones_v, acc.at[dst_v.at[j]], add=True)

        plsc.subcore_barrier()
        pltpu.sync_copy(acc.at[pl.ds(sid * RPT, RPT)],
                        out_hbm.at[cid, pl.ds(sid * RPT, RPT)])

    return k(dst4, ones_h, zeros_h)


def _sc_agg(table, src_idx, dst_idx, zeros_h, ch):
    @functools.partial(
        pl.kernel, mesh=_mesh(),
        out_type=jax.ShapeDtypeStruct((NC, NP, HALF), jnp.float32),
        scratch_types=[pltpu.VMEM((CHUNK,), jnp.int32) for _ in range(NBUF)]
        + [pltpu.VMEM((CHUNK,), jnp.int32) for _ in range(NBUF)]
        + [pltpu.VMEM((CHUNK, HALF), jnp.float32) for _ in range(NBUF)]
        + [pltpu.SemaphoreType.DMA for _ in range(3 * NBUF)]
        + [pltpu.VMEM_SHARED((N_ACC, HALF), jnp.float32)],
    )
    def k(tab_hbm, src_hbm, dst_hbm, zeros_hbm, out_hbm, *rest):
        sidx = rest[:NBUF]
        didx = rest[NBUF:2 * NBUF]
        rows = rest[2 * NBUF:3 * NBUF]
        sis = rest[3 * NBUF:4 * NBUF]
        sds = rest[4 * NBUF:5 * NBUF]
        sgs = rest[5 * NBUF:6 * NBUF]
        acc = rest[6 * NBUF]
        cid = lax.axis_index("c")
        sid = lax.axis_index("s")
        pltpu.sync_copy(zeros_hbm, acc.at[pl.ds(sid * RPT_A, RPT_A)])
        plsc.subcore_barrier()

        for b in range(NBUF):
            pltpu.async_copy(src_hbm.at[cid, sid, b], sidx[b], sis[b])
            pltpu.async_copy(dst_hbm.at[cid, sid, b], didx[b], sds[b])
        for b in range(NBUF):
            pltpu.make_async_copy(src_hbm.at[cid, sid, 0], sidx[b], sis[b]).wait()
            pltpu.async_copy(tab_hbm.at[sidx[b]], rows[b], sgs[b])

        @pl.loop(0, ch // NBUF)
        def _(g):
            for b in range(NBUF):
                j = g * NBUF + b
                nxt = j + NBUF
                pltpu.make_async_copy(tab_hbm.at[sidx[b]], rows[b],
                                      sgs[b]).wait()

                @pl.when(nxt < ch)
                def _():
                    pltpu.async_copy(src_hbm.at[cid, sid, nxt], sidx[b], sis[b])

                pltpu.make_async_copy(dst_hbm.at[cid, sid, 0], didx[b],
                                      sds[b]).wait()
                pltpu.sync_copy(rows[b], acc.at[didx[b]], add=True)

                @pl.when(nxt < ch)
                def _():
                    pltpu.async_copy(dst_hbm.at[cid, sid, nxt], didx[b], sds[b])
                    pltpu.make_async_copy(src_hbm.at[cid, sid, 0], sidx[b],
                                          sis[b]).wait()
                    pltpu.async_copy(tab_hbm.at[sidx[b]], rows[b], sgs[b])

        plsc.subcore_barrier()
        pltpu.sync_copy(acc.at[pl.ds(sid * RPT_A, RPT_A)],
                        out_hbm.at[cid, pl.ds(sid * RPT_A, RPT_A)])

    return k(table, src_idx, dst_idx, zeros_h)


def _dinv(deg_blk):
    d = deg_blk[0, :, :1] + deg_blk[1, :, :1] + 1.0
    return lax.rsqrt(d)


def _tc_xws(xp, W1, deg):
    def body(x_ref, w_ref, deg_ref, out_ref):
        xw = lax.dot_general(x_ref[...], w_ref[...], (((1,), (0,)), ((), ())),
                             precision=lax.Precision.HIGHEST)
        xws = xw * _dinv(deg_ref[...])
        out_ref[0] = xws[:, :HALF]
        out_ref[1] = xws[:, HALF:]

    return pl.pallas_call(
        body,
        grid=(NP // BLK,),
        in_specs=[pl.BlockSpec((BLK, D), lambda i: (i, 0)),
                  pl.BlockSpec((D, D), lambda i: (0, 0)),
                  pl.BlockSpec((NC, BLK, HALF), lambda i: (0, i, 0))],
        out_specs=pl.BlockSpec((NC, BLK, HALF), lambda i: (0, i, 0)),
        out_shape=jax.ShapeDtypeStruct((NC, NP, HALF), jnp.float32),
    )(xp, W1, deg)


def _tc_hidden(agg, xws, deg, b1r, W2p):
    def body(agg_ref, xws_ref, deg_ref, b1_ref, w2_ref, out_ref):
        a = agg_ref[...]
        s = xws_ref[...]
        full_a = jnp.concatenate([a[0], a[1]], axis=1)
        full_s = jnp.concatenate([s[0], s[1]], axis=1)
        dinv = _dinv(deg_ref[...])
        h = jnp.maximum(dinv * (full_a + full_s) + b1_ref[0], 0.0)
        hw = lax.dot_general(h, w2_ref[...], (((1,), (0,)), ((), ())),
                             precision=lax.Precision.HIGHEST)
        out_ref[...] = dinv * hw

    return pl.pallas_call(
        body,
        grid=(NP // BLK,),
        in_specs=[pl.BlockSpec((NC, BLK, HALF), lambda i: (0, i, 0)),
                  pl.BlockSpec((NC, BLK, HALF), lambda i: (0, i, 0)),
                  pl.BlockSpec((NC, BLK, HALF), lambda i: (0, i, 0)),
                  pl.BlockSpec((1, D), lambda i: (0, 0)),
                  pl.BlockSpec((D, HALF), lambda i: (0, 0))],
        out_specs=pl.BlockSpec((BLK, HALF), lambda i: (i, 0)),
        out_shape=jax.ShapeDtypeStruct((NP, HALF), jnp.float32),
    )(agg, xws, deg, b1r, W2p)


def _tc_final(agg2, hws, deg, b2p):
    def body(agg_ref, hws_ref, deg_ref, b2_ref, out_ref):
        a = agg_ref[...]
        dinv = _dinv(deg_ref[...])
        v = dinv * (a[0] + a[1] + hws_ref[...]) + b2_ref[0]
        col = lax.broadcasted_iota(jnp.int32, (BLK, HALF), 1)
        valid = col < NCLS
        m = jnp.max(jnp.where(valid, v, -1e30), axis=1, keepdims=True)
        ex = jnp.where(valid, jnp.exp(v - m), 0.0)
        ssum = jnp.sum(ex, axis=1, keepdims=True)
        out_ref[...] = v - m - jnp.log(ssum)

    return pl.pallas_call(
        body,
        grid=(NP // BLK,),
        in_specs=[pl.BlockSpec((NC, BLK, HALF), lambda i: (0, i, 0)),
                  pl.BlockSpec((BLK, HALF), lambda i: (i, 0)),
                  pl.BlockSpec((NC, BLK, HALF), lambda i: (0, i, 0)),
                  pl.BlockSpec((1, HALF), lambda i: (0, 0))],
        out_specs=pl.BlockSpec((BLK, HALF), lambda i: (i, 0)),
        out_shape=jax.ShapeDtypeStruct((NP, HALF), jnp.float32),
    )(agg2, hws, deg, b2p)


def kernel(x, edge_index, W1, b1, W2, b2):
    src = edge_index[0].astype(jnp.int32)
    dst = edge_index[1].astype(jnp.int32)
    e = src.shape[0]
    pad = E_PAD - e
    src_p = jnp.concatenate([src, jnp.zeros((pad,), jnp.int32)])
    dst_p = jnp.concatenate([dst, jnp.full((pad,), DUMP, jnp.int32)])

    dst4 = dst_p.reshape(NW, CH_D, DCHUNK)
    src_w = src_p.reshape(NC, NS, CH_W, CHUNK)
    dst_w = dst_p.reshape(NC, NS, CH_W, CHUNK)
    src_t = jnp.stack([src_p, src_p + NP]).reshape(NC, NS, CH_T, CHUNK)
    dst_t = jnp.broadcast_to(dst_p.reshape(1, NS, CH_T, CHUNK),
                             (NC, NS, CH_T, CHUNK))

    xp = jnp.pad(x, ((0, NP - N), (0, 0)))
    zeros_h = jnp.zeros((RPT, HALF), jnp.float32)
    zeros_a = jnp.zeros((RPT_A, HALF), jnp.float32)
    ones_h = jnp.ones((DCHUNK, HALF), jnp.float32)
    b1r = b1.reshape(1, D)
    W2p = jnp.pad(W2, ((0, 0), (0, HALF - NCLS)))
    b2p = jnp.pad(b2, (0, HALF - NCLS)).reshape(1, HALF)

    deg = _sc_deg(dst4, ones_h, zeros_h)
    xws = _tc_xws(xp, W1, deg)
    table = xws.reshape(NC * NP, HALF)
    agg = _sc_agg(table, src_t, dst_t, zeros_a, CH_T)
    hws = _tc_hidden(agg, xws, deg, b1r, W2p)
    agg2 = _sc_agg(hws, src_w, dst_w, zeros_a, CH_W)
    outw = _tc_final(agg2, hws, deg, b2p)
    return outw[:N, :NCLS]
